# baseline (speedup 1.0000x reference)
import functools
import math

import jax
import jax.numpy as jnp
from jax import lax
from jax.experimental import layout as jex_layout
from jax.experimental import pallas as pl
from jax.experimental.pallas import tpu as pltpu
from jax.experimental.pallas import tpu_sc as plsc

_NUM_CORES = 2
_NUM_SUBCORES = 16
_NW = _NUM_CORES * _NUM_SUBCORES
_LANES = 16


def _positional_encoding(max_len, d_model):
    pos = jnp.arange(max_len, dtype=jnp.float32)[:, None]
    index = jnp.arange(d_model, dtype=jnp.float32)[None, :]
    pe = pos / jnp.power(10000.0, (index - index % 2) / float(d_model))
    pe_s = jnp.sin(pe[:, 0::2])[..., None]
    pe_c = jnp.cos(pe[:, 1::2])[..., None]
    return jnp.concatenate([pe_s, pe_c], axis=-1).reshape(pe.shape[0], -1)


@functools.partial(jax.jit, static_argnames=("seq_len", "d"))
def _lookup(idx_w, h_off, table_wide, pe_flat, seq_len, d):
    (n,) = idx_w.shape
    per_w = n // _NW
    ch = seq_len
    n_ch = per_w // ch
    wd = 2 * d
    sub = ((0, 104), (104, 96))
    scale = float(math.sqrt(d))
    mesh = plsc.VectorSubcoreMesh(core_axis_name="c", subcore_axis_name="s")

    @functools.partial(
        pl.kernel,
        out_type=jax.ShapeDtypeStruct((n * d,), jnp.float32),
        mesh=mesh,
        compiler_params=pltpu.CompilerParams(use_tc_tiling_on_sc=True),
        scratch_types=[
            pltpu.VMEM((per_w,), jnp.int32),
            pltpu.VMEM((per_w + _LANES,), jnp.int32),
            pltpu.VMEM((2, ch, wd), jnp.float32),
            pltpu.VMEM((2, ch * d), jnp.float32),
            pltpu.VMEM((seq_len * d,), jnp.float32),
            [pltpu.SemaphoreType.DMA] * 2,
            [pltpu.SemaphoreType.DMA] * 2,
        ],
    )
    def k(tab_hbm, idx_hbm, h_hbm, pe_hbm, out_hbm,
          idx_v, h_v, wide_v, out_v, pe_v, gsems, ssems):
        wid = lax.axis_index("s") * _NUM_CORES + lax.axis_index("c")
        pltpu.sync_copy(pe_hbm, pe_v)
        pltpu.sync_copy(idx_hbm.at[pl.ds(wid * per_w, per_w)], idx_v)
        pltpu.sync_copy(h_hbm.at[pl.ds(wid * per_w, per_w)],
                        h_v.at[pl.ds(0, per_w)])

        def _gather_copies(c, b):
            return [
                pltpu.make_async_copy(
                    tab_hbm.at[idx_v.at[pl.ds(c * ch + off, klen)]],
                    wide_v.at[b].at[pl.ds(off, klen)],
                    gsems[b],
                )
                for off, klen in sub
            ]

        def start_gather(c, b):
            for cp in _gather_copies(c, b):
                cp.start()

        def wait_gather(c, b):
            for cp in _gather_copies(c, b):
                cp.wait()

        def start_scatter(c, b):
            base = (wid * per_w + c * ch) * d
            pltpu.async_copy(out_v.at[b], out_hbm.at[pl.ds(base, ch * d)],
                             ssems[b])

        def wait_scatter(b):
            pltpu.make_async_copy(
                out_v.at[b], out_hbm.at[pl.ds(0, ch * d)], ssems[b]
            ).wait()

        def compute(c, b):
            wrow = wide_v.at[b]
            obuf = out_v.at[b]

            @plsc.parallel_loop(0, ch, unroll=2)
            def _(r):
                h = h_v[pl.ds(c * ch + r, _LANES)][0]
                for t in range(d // _LANES):
                    o = r * d + t * _LANES
                    x = wrow[r, pl.ds(h + t * _LANES, _LANES)]
                    obuf[pl.ds(o, _LANES)] = (
                        x * scale + pe_v[pl.ds(r * d + t * _LANES, _LANES)]
                    )

        start_gather(0, 0)

        def outer(o, carry):
            for bb in range(2):
                c = o * 2 + bb
                nxt = 1 - bb

                @pl.when(c + 1 < n_ch)
                def _():
                    @pl.when(c >= 1)
                    def _():
                        wait_scatter(nxt)

                    start_gather(c + 1, nxt)

                wait_gather(c, bb)
                compute(c, bb)
                start_scatter(c, bb)
            return carry

        lax.fori_loop(0, n_ch // 2, outer, 0)
        wait_scatter(0)
        wait_scatter(1)

    return k(table_wide, idx_w, h_off, pe_flat)


def kernel(sequences, table):
    b, s = sequences.shape
    v, d = table.shape
    n = b * s
    idx = sequences.astype(jnp.int32).reshape(n)
    idx_w = idx >> 1
    h_off = (idx & 1) << 6
    table_wide = table.reshape(v // 2, 2 * d)
    table_wide = jex_layout.with_layout_constraint(
        table_wide,
        jex_layout.Layout(major_to_minor=(0, 1), tiling=((8, 128),)),
    )
    pe_flat = _positional_encoding(s, d).reshape(s * d)
    out = _lookup(idx_w, h_off, table_wide, pe_flat, s, d)
    return out.reshape(b, s, d)


# speedup vs baseline: 1.0475x; 1.0475x over previous
import functools
import math

import jax
import jax.numpy as jnp
from jax import lax
from jax.experimental import pallas as pl
from jax.experimental.pallas import tpu as pltpu
from jax.experimental.pallas import tpu_sc as plsc

_NUM_CORES = 2
_NUM_SUBCORES = 16
_NW = _NUM_CORES * _NUM_SUBCORES
_LANES = 16
_NBUF = 4
_SEQ_PER_CHUNK = 2


def _positional_encoding(max_len, d_model):
    pos = jnp.arange(max_len, dtype=jnp.float32)[:, None]
    index = jnp.arange(d_model, dtype=jnp.float32)[None, :]
    pe = pos / jnp.power(10000.0, (index - index % 2) / float(d_model))
    pe_s = jnp.sin(pe[:, 0::2])[..., None]
    pe_c = jnp.cos(pe[:, 1::2])[..., None]
    return jnp.concatenate([pe_s, pe_c], axis=-1).reshape(pe.shape[0], -1)


@functools.partial(jax.jit, static_argnames=("v", "seq_len", "d"))
def _lookup(idx, table_flat, pe_flat, v, seq_len, d):
    (n,) = idx.shape
    per_w = n // _NW
    ch = _SEQ_PER_CHUNK * seq_len
    n_ch = per_w // ch
    subs = []
    off = 0
    while off < ch:
        klen = min(104, ch - off)
        subs.append((off, klen))
        off += klen
    scale = float(math.sqrt(d))
    mesh = plsc.VectorSubcoreMesh(core_axis_name="c", subcore_axis_name="s")

    @functools.partial(
        pl.kernel,
        out_type=jax.ShapeDtypeStruct((n, d), jnp.float32),
        mesh=mesh,
        compiler_params=pltpu.CompilerParams(use_tc_tiling_on_sc=False),
        scratch_types=[
            pltpu.VMEM((per_w,), jnp.int32),
            pltpu.VMEM((_NBUF, ch, d), jnp.float32),
            pltpu.VMEM((seq_len * d,), jnp.float32),
            [pltpu.SemaphoreType.DMA] * _NBUF,
            [pltpu.SemaphoreType.DMA] * _NBUF,
        ],
    )
    def k(tab_hbm, idx_hbm, pe_hbm, out_hbm,
          idx_v, rows_v, pe_v, gsems, ssems):
        wid = lax.axis_index("s") * _NUM_CORES + lax.axis_index("c")
        tab2d = tab_hbm
        out2d = out_hbm
        pltpu.sync_copy(pe_hbm, pe_v)
        pltpu.sync_copy(idx_hbm.at[pl.ds(wid * per_w, per_w)], idx_v)

        def _gather_copies(c, b):
            return [
                pltpu.make_async_copy(
                    tab2d.at[idx_v.at[pl.ds(c * ch + o, klen)]],
                    rows_v.at[b].at[pl.ds(o, klen)],
                    gsems[b],
                )
                for o, klen in subs
            ]

        def start_gather(c, b):
            for cp in _gather_copies(c, b):
                cp.start()

        def wait_gather(c, b):
            for cp in _gather_copies(c, b):
                cp.wait()

        def start_scatter(c, b):
            row0 = wid * per_w + c * ch
            pltpu.async_copy(rows_v.at[b], out2d.at[pl.ds(row0, ch)], ssems[b])

        def wait_scatter(b):
            pltpu.make_async_copy(
                rows_v.at[b], out2d.at[pl.ds(0, ch)], ssems[b]
            ).wait()

        def compute(b):
            buf = rows_v.at[b]

            @plsc.parallel_loop(0, seq_len, unroll=2)
            def _(p):
                for t in range(d // _LANES):
                    sl = pl.ds(t * _LANES, _LANES)
                    pe_val = pe_v[pl.ds(p * d + t * _LANES, _LANES)]
                    buf[p, sl] = buf[p, sl] * scale + pe_val
                    q = p + seq_len
                    buf[q, sl] = buf[q, sl] * scale + pe_val

        start_gather(0, 0)
        start_gather(1, 1)

        def outer(o, carry):
            for bb in range(_NBUF):
                c = o * _NBUF + bb
                bn = (bb + 2) % _NBUF

                @pl.when(c + 2 < n_ch)
                def _():
                    @pl.when(c >= 2)
                    def _():
                        wait_scatter(bn)

                    start_gather(c + 2, bn)

                wait_gather(c, bb)
                compute(bb)
                start_scatter(c, bb)
            return carry

        lax.fori_loop(0, n_ch // _NBUF, outer, 0)
        wait_scatter((n_ch - 2) % _NBUF)
        wait_scatter((n_ch - 1) % _NBUF)

    table_flat = lax.optimization_barrier(table_flat)
    table2d = table_flat.reshape(v, d)
    return k(table2d, idx, pe_flat)


def kernel(sequences, table):
    b, s = sequences.shape
    v, d = table.shape
    n = b * s
    idx = sequences.astype(jnp.int32).reshape(n)
    table_flat = table.reshape(v * d)
    pe_flat = _positional_encoding(s, d).reshape(s * d)
    out = _lookup(idx, table_flat, pe_flat, v, s, d)
    return out.reshape(b, s, d)
